# Initial kernel scaffold; baseline (speedup 1.0000x reference)
import jax
import jax.numpy as jnp
from jax.experimental import pallas as pl


def kernel(token_ids, token_emb_table, token_type_emb_table, full_position_emb_table):
    raise NotImplementedError("write your pallas kernel here")


# speedup vs baseline: 2.6126x; 2.6126x over previous
import functools

import jax
import jax.numpy as jnp
from jax import lax
from jax.experimental import pallas as pl
from jax.experimental.pallas import tpu as pltpu
from jax.experimental.pallas import tpu_sc as plsc

SEP = 102
DIM = 128
NC, NS = 2, 16
NW = NC * NS
W = 128
LANES = 16


def _prep_body(seq, ids_ref, tt_ref, pos_ref, cidx_ref, add_ref):
    ids = ids_ref[...]
    sep = (ids == SEP).astype(jnp.int32)
    c = sep
    sh = 1
    while sh < seq:
        z = jnp.zeros((ids.shape[0], sh), jnp.int32)
        c = c + jnp.concatenate([z, c[:, : seq - sh]], axis=1)
        sh *= 2
    seg = jnp.minimum(c - sep, 2)
    col = lax.broadcasted_iota(jnp.int32, ids.shape, 1)
    cidx_ref[...] = col + seq * seg
    pos = pos_ref[:seq, :]
    add_ref[:seq, :] = pos + tt_ref[0:1, :]
    add_ref[seq : 2 * seq, :] = pos + tt_ref[1:2, :]
    add_ref[2 * seq : 3 * seq, :] = pos


def _gather_body(nwin, table_hbm, addend_hbm, tid_hbm, cidx_hbm, out_hbm,
                 tid_v, cid_v, row_v, add_v, sem_t, sem_a):
    wid = lax.axis_index("s") * NC + lax.axis_index("c")

    @pl.loop(0, nwin)
    def _(w):
        base = (wid * nwin + w) * W
        pltpu.sync_copy(tid_hbm.at[pl.ds(base, W)], tid_v)
        pltpu.sync_copy(cidx_hbm.at[pl.ds(base, W)], cid_v)
        g_t = pltpu.async_copy(table_hbm.at[tid_v], row_v, sem_t)
        g_a = pltpu.async_copy(addend_hbm.at[cid_v], add_v, sem_a)
        g_t.wait()
        g_a.wait()

        @pl.loop(0, W)
        def _(r):
            for ch in range(DIM // LANES):
                slc = (pl.ds(r, 1), pl.ds(ch * LANES, LANES))
                row_v.at[slc][...] = row_v.at[slc][...] + add_v.at[slc][...]

        pltpu.sync_copy(row_v, out_hbm.at[pl.ds(base, W)])


def kernel(token_ids, token_emb_table, token_type_emb_table, full_position_emb_table):
    batch, seq = token_ids.shape
    token_ids = token_ids.astype(jnp.int32)

    cidx, addend = pl.pallas_call(
        functools.partial(_prep_body, seq),
        out_shape=[
            jax.ShapeDtypeStruct((batch, seq), jnp.int32),
            jax.ShapeDtypeStruct((3 * seq, DIM), jnp.float32),
        ],
    )(token_ids, token_type_emb_table, full_position_emb_table)

    total = batch * seq
    chunk = NW * W
    padded = ((total + chunk - 1) // chunk) * chunk
    tid_flat = token_ids.reshape(-1)
    cid_flat = cidx.reshape(-1)
    if padded != total:
        pad = padded - total
        tid_flat = jnp.pad(tid_flat, (0, pad))
        cid_flat = jnp.pad(cid_flat, (0, pad))
    nwin = padded // chunk

    mesh = plsc.VectorSubcoreMesh(core_axis_name="c", subcore_axis_name="s")
    out = pl.kernel(
        functools.partial(_gather_body, nwin),
        out_type=jax.ShapeDtypeStruct((padded, DIM), jnp.float32),
        mesh=mesh,
        scratch_types=[
            pltpu.VMEM((W,), jnp.int32),
            pltpu.VMEM((W,), jnp.int32),
            pltpu.VMEM((W, DIM), jnp.float32),
            pltpu.VMEM((W, DIM), jnp.float32),
            pltpu.SemaphoreType.DMA,
            pltpu.SemaphoreType.DMA,
        ],
    )(token_emb_table, addend, tid_flat, cid_flat)
    if padded != total:
        out = out[:total]
    return out.reshape(batch, seq, DIM)
